# baseline (speedup 1.0000x reference)
import functools

import jax
import jax.numpy as jnp
from jax import lax
from jax.experimental import pallas as pl
from jax.experimental.pallas import tpu as pltpu
from jax.experimental.pallas import tpu_sc as plsc

N_DEV = 10000
DEG = 16
N_BRE = 80000
EMB = 128

NW = 32
PER_W = 320
N_PAD = NW * PER_W
CH = 80
N_CH = PER_W // CH
NJ = DEG * N_CH

TC_BLK = 256


def _sc_body(pkt_hbm, bs_hbm, devt_hbm, vpre_hbm, ne_hbm, cbs_hbm,
             pkt_v, dev_r, tbs, nb_r, acc, sem_d, sem_b, sem_v):
    wid = lax.axis_index("s") * 2 + lax.axis_index("c")
    base = wid * PER_W

    pltpu.sync_copy(pkt_hbm, pkt_v)
    pltpu.sync_copy(devt_hbm.at[wid, pl.ds(0, N_CH)], dev_r.at[pl.ds(0, N_CH)])

    iota16 = lax.iota(jnp.int32, 16)

    def fire_dev(d_next):
        src_row = jnp.minimum(d_next, DEG - 1) * N_CH
        par = (d_next % 2) * N_CH
        pltpu.async_copy(devt_hbm.at[wid, pl.ds(src_row, N_CH)],
                         dev_r.at[pl.ds(par, N_CH)], sem_d)

    def drain_dev():
        pltpu.make_async_copy(devt_hbm.at[0, pl.ds(0, N_CH)],
                              dev_r.at[pl.ds(0, N_CH)], sem_d).wait()

    def compute(d):
        par = (d % 2) * N_CH
        for ci in range(N_CH):
            for g in range(CH // 16):
                sl = pl.ds(g * 16, 16)
                devv = dev_r[par + ci, sl]
                pk = plsc.load_gather(pkt_v, [devv])
                e0 = pk & 16383
                e1 = pk >> 14
                nid = base + ci * CH + g * 16 + iota16
                nb_r[par + ci, sl] = jnp.where(e0 != nid, e0, e1)

    def fire_bs(d):
        par = (d % 2) * N_CH
        for ci in range(N_CH):
            pltpu.async_copy(bs_hbm.at[dev_r.at[par + ci]],
                             tbs.at[d * N_CH + ci], sem_b)

    def drain_bs():
        for _ in range(N_CH):
            pltpu.make_async_copy(
                bs_hbm.at[pl.ds(0, CH)], tbs.at[0], sem_b).wait()

    def fire_vpre(d, add):
        par = (d % 2) * N_CH
        for ci in range(N_CH):
            pltpu.async_copy(
                vpre_hbm.at[nb_r.at[par + ci]],
                acc.at[pl.ds(ci * CH, CH)], sem_v, add=add)

    def drain_vpre():
        for ci in range(N_CH):
            pltpu.make_async_copy(
                vpre_hbm.at[pl.ds(0, CH)],
                acc.at[pl.ds(ci * CH, CH)], sem_v).wait()

    fire_dev(1)
    compute(0)
    fire_bs(0)
    fire_vpre(0, add=False)

    def d_body(d, carry):
        drain_dev()
        fire_dev(d + 1)
        compute(d)
        drain_bs()
        fire_bs(d)
        drain_vpre()
        fire_vpre(d, add=True)
        return carry

    lax.fori_loop(1, DEG, d_body, 0, unroll=False)
    drain_dev()
    drain_bs()
    drain_vpre()

    pltpu.sync_copy(acc, ne_hbm.at[pl.ds(base, PER_W)])
    pltpu.sync_copy(tbs, cbs_hbm.at[wid])


@jax.jit
def _sc_gather(pkt, bst, devt, vpre):
    mesh = plsc.VectorSubcoreMesh(core_axis_name="c", subcore_axis_name="s")
    fn = functools.partial(
        pl.kernel,
        out_type=(
            jax.ShapeDtypeStruct((N_PAD, EMB), jnp.float32),
            jax.ShapeDtypeStruct((NW, NJ, CH), jnp.float32),
        ),
        mesh=mesh,
        compiler_params=pltpu.CompilerParams(
            needs_layout_passes=False, use_tc_tiling_on_sc=False),
        scratch_types=[
            pltpu.VMEM((N_BRE,), jnp.int32),
            pltpu.VMEM((2 * N_CH, CH), jnp.int32),
            pltpu.VMEM((NJ, CH), jnp.float32),
            pltpu.VMEM((2 * N_CH, CH), jnp.int32),
            pltpu.VMEM((PER_W, EMB), jnp.float32),
            pltpu.SemaphoreType.DMA,
            pltpu.SemaphoreType.DMA,
            pltpu.SemaphoreType.DMA,
        ],
    )(_sc_body)
    return fn(pkt, bst, devt, vpre)


def _tc_body(ne_ref, cbs_ref, ps_ref, w0t, w1r, w2r, w3t, w4r, w5t,
             bias, wcb, out_ref):
    cbs = cbs_ref[...]
    ps = ps_ref[...]
    ne = ne_ref[...]

    b0r = bias[0:1, :]
    b1r = bias[1:2, :]
    b2r = bias[2:3, :]
    b3r = bias[3:4, :]
    b4r = bias[4:5, :]
    b5r = bias[5:6, :]

    w4 = w4r[...]
    be = jnp.tanh(cbs[:, 0:1] * w4 + b4r)
    for d in range(1, DEG):
        be = be + jnp.tanh(cbs[:, d:d + 1] * w4 + b4r)
    breaker = jnp.tanh(
        jnp.dot(be, w3t[...], preferred_element_type=jnp.float32) + b3r)

    tmp = jnp.sum(cbs, axis=1, keepdims=True)
    w1 = w1r[...]
    pe = jnp.tanh(ps[:, 0:1] * w1 + b1r)
    for i in range(1, 3):
        pe = pe + jnp.tanh(ps[:, i:i + 1] * w1 + b1r)
    pe = pe + 3.0 * jnp.tanh(tmp * w2r[...] + b2r)
    protector = jnp.tanh(
        jnp.dot(pe, w0t[...], preferred_element_type=jnp.float32) + b0r)

    neighbor = jnp.tanh(
        jnp.dot(ne, w5t[...], preferred_element_type=jnp.float32) + b5r)

    wc = wcb[...]
    out_ref[...] = jnp.tanh(
        protector * wc[0:1, :] + breaker * wc[1:2, :]
        + neighbor * wc[2:3, :] + wc[3:4, :])


@jax.jit
def _tc_dense(ne, cbs, ps, w0t, w1r, w2r, w3t, w4r, w5t, bias, wcb):
    grid = (N_PAD // TC_BLK,)
    full = lambda shape: pl.BlockSpec(shape, lambda i: (0, 0))
    return pl.pallas_call(
        _tc_body,
        grid=grid,
        in_specs=[
            pl.BlockSpec((TC_BLK, EMB), lambda i: (i, 0)),
            pl.BlockSpec((TC_BLK, DEG), lambda i: (i, 0)),
            pl.BlockSpec((TC_BLK, 4), lambda i: (i, 0)),
            full((EMB, EMB)), full((1, EMB)), full((1, EMB)),
            full((EMB, EMB)), full((1, EMB)), full((EMB, EMB)),
            full((8, EMB)), full((8, EMB)),
        ],
        out_specs=pl.BlockSpec((TC_BLK, EMB), lambda i: (i, 0)),
        out_shape=jax.ShapeDtypeStruct((N_PAD, EMB), jnp.float32),
    )(ne, cbs, ps, w0t, w1r, w2r, w3t, w4r, w5t, bias, wcb)


def kernel(V_pre, devices, breakers, protector_sate, breaker_state,
           W0, b0, W1, b1, W2, b2, W3, b3, W4, b4, W5, b5, Wc, bc):
    dev = devices.astype(jnp.int32)
    br = breakers.astype(jnp.int32)
    pkt = br[:, 0] | (br[:, 1] << 14)

    dev_p = jnp.pad(dev, ((0, N_PAD - N_DEV), (0, 0)))
    devt = (dev_p.T.reshape(DEG, NW, N_CH, CH)
            .transpose(1, 0, 2, 3).reshape(NW, NJ, CH))
    ps_p = jnp.pad(protector_sate, ((0, N_PAD - N_DEV), (0, 1)))

    ne, cbs3 = _sc_gather(pkt, breaker_state, devt, V_pre)
    cbs = (cbs3.reshape(NW, DEG, N_CH, CH)
           .transpose(0, 2, 3, 1).reshape(N_PAD, DEG))

    row = lambda v: v.reshape(1, EMB)
    bias = jnp.concatenate(
        [row(b0), row(b1), row(b2), row(b3), row(b4), row(b5),
         jnp.zeros((2, EMB), jnp.float32)], axis=0)
    wcb = jnp.concatenate(
        [jnp.broadcast_to(Wc[0], (1, EMB)), jnp.broadcast_to(Wc[1], (1, EMB)),
         jnp.broadcast_to(Wc[2], (1, EMB)), jnp.broadcast_to(bc[0], (1, EMB)),
         jnp.zeros((4, EMB), jnp.float32)], axis=0)

    out = _tc_dense(ne, cbs, ps_p, W0.T, W1.T, W2.T, W3.T, W4.T, W5.T,
                    bias, wcb)
    return out[:N_DEV]


# speedup vs baseline: 1.0829x; 1.0829x over previous
import functools

import jax
import jax.numpy as jnp
from jax import lax
from jax.experimental import pallas as pl
from jax.experimental.pallas import tpu as pltpu
from jax.experimental.pallas import tpu_sc as plsc

N_DEV = 10000
DEG = 16
N_BRE = 80000
EMB = 128

NW = 32
PER_W = 320
N_PAD = NW * PER_W
CH = 80
N_CH = PER_W // CH
NJ = DEG * N_CH

TC_BLK = 256


def _sc_body(pkt_hbm, bs_hbm, devt_hbm, vpre_hbm, ne_hbm, cbs_hbm,
             dev_t, tpk, tbs, nb_v, stage, acc,
             sem_pk, sem_b, sem_v, sem_a):
    wid = lax.axis_index("s") * 2 + lax.axis_index("c")
    base = wid * PER_W

    pltpu.sync_copy(devt_hbm.at[wid], dev_t)

    iota16 = lax.iota(jnp.int32, 16)

    def fire_pkt(d):
        for ci in range(N_CH):
            j = d * N_CH + ci
            pltpu.async_copy(pkt_hbm.at[dev_t.at[j]], tpk.at[j], sem_pk)

    def drain_pkt():
        for _ in range(N_CH):
            pltpu.make_async_copy(
                pkt_hbm.at[pl.ds(0, CH)], tpk.at[0], sem_pk).wait()

    def fire_bs(d):
        for ci in range(N_CH):
            j = d * N_CH + ci
            pltpu.async_copy(bs_hbm.at[dev_t.at[j]], tbs.at[j], sem_b)

    def compute_nb(d):
        for ci in range(N_CH):
            j = d * N_CH + ci
            for g in range(CH // 16):
                sl = pl.ds(g * 16, 16)
                pk = tpk[j, sl]
                e0 = pk & 16383
                e1 = pk >> 14
                nid = base + ci * CH + g * 16 + iota16
                nb_v[j, sl] = jnp.where(e0 != nid, e0, e1)

    def fire_stage(d, ci):
        pltpu.async_copy(
            vpre_hbm.at[nb_v.at[d * N_CH + ci]],
            stage.at[pl.ds(ci * CH, CH)], sem_v)

    def drain_stage():
        pltpu.make_async_copy(
            vpre_hbm.at[pl.ds(0, CH)],
            stage.at[pl.ds(0, CH)], sem_v).wait()

    def acc_add(ci):
        def row_body(r, carry):
            for k in range(EMB // 16):
                sl = pl.ds(k * 16, 16)
                acc[r, sl] = acc[r, sl] + stage[r, sl]
            return carry
        lax.fori_loop(ci * CH, ci * CH + CH, row_body, 0, unroll=4)

    fire_pkt(0)
    fire_pkt(1)
    drain_pkt()
    compute_nb(0)
    for ci in range(N_CH):
        pltpu.async_copy(
            vpre_hbm.at[nb_v.at[ci]],
            acc.at[pl.ds(ci * CH, CH)], sem_a)
    fire_bs(0)
    fire_pkt(2)
    drain_pkt()
    compute_nb(1)
    for ci in range(N_CH):
        fire_stage(1, ci)
    fire_bs(1)
    fire_pkt(3)
    for ci in range(N_CH):
        pltpu.make_async_copy(
            vpre_hbm.at[pl.ds(0, CH)],
            acc.at[pl.ds(ci * CH, CH)], sem_a).wait()

    def d_body(d, carry):
        drain_pkt()
        compute_nb(d + 1)
        fire_pkt(jnp.minimum(d + 3, DEG - 1))
        fire_bs(d + 1)
        for ci in range(N_CH):
            drain_stage()
            acc_add(ci)
            fire_stage(d + 1, ci)
        return carry

    lax.fori_loop(1, DEG - 1, d_body, 0, unroll=False)

    for ci in range(N_CH):
        drain_stage()
        acc_add(ci)
    drain_pkt()
    drain_pkt()

    def bs_drain_body(i, carry):
        for _ in range(N_CH):
            pltpu.make_async_copy(
                bs_hbm.at[pl.ds(0, CH)], tbs.at[0], sem_b).wait()
        return carry
    lax.fori_loop(0, DEG, bs_drain_body, 0, unroll=False)

    pltpu.sync_copy(acc, ne_hbm.at[pl.ds(base, PER_W)])
    pltpu.sync_copy(tbs, cbs_hbm.at[wid])


@jax.jit
def _sc_gather(pkt, bst, devt, vpre):
    mesh = plsc.VectorSubcoreMesh(core_axis_name="c", subcore_axis_name="s")
    fn = functools.partial(
        pl.kernel,
        out_type=(
            jax.ShapeDtypeStruct((N_PAD, EMB), jnp.float32),
            jax.ShapeDtypeStruct((NW, NJ, CH), jnp.float32),
        ),
        mesh=mesh,
        compiler_params=pltpu.CompilerParams(
            needs_layout_passes=False, use_tc_tiling_on_sc=False),
        scratch_types=[
            pltpu.VMEM((NJ, CH), jnp.int32),
            pltpu.VMEM((NJ, CH), jnp.int32),
            pltpu.VMEM((NJ, CH), jnp.float32),
            pltpu.VMEM((NJ, CH), jnp.int32),
            pltpu.VMEM((PER_W, EMB), jnp.float32),
            pltpu.VMEM((PER_W, EMB), jnp.float32),
            pltpu.SemaphoreType.DMA,
            pltpu.SemaphoreType.DMA,
            pltpu.SemaphoreType.DMA,
            pltpu.SemaphoreType.DMA,
        ],
    )(_sc_body)
    return fn(pkt, bst, devt, vpre)


def _tc_body(ne_ref, cbs_ref, ps_ref, w0t, w1r, w2r, w3t, w4r, w5t,
             bias, wcb, out_ref):
    cbs = cbs_ref[...]
    ps = ps_ref[...]
    ne = ne_ref[...]

    b0r = bias[0:1, :]
    b1r = bias[1:2, :]
    b2r = bias[2:3, :]
    b3r = bias[3:4, :]
    b4r = bias[4:5, :]
    b5r = bias[5:6, :]

    w4 = w4r[...]
    be = jnp.tanh(cbs[:, 0:1] * w4 + b4r)
    for d in range(1, DEG):
        be = be + jnp.tanh(cbs[:, d:d + 1] * w4 + b4r)
    breaker = jnp.tanh(
        jnp.dot(be, w3t[...], preferred_element_type=jnp.float32) + b3r)

    tmp = jnp.sum(cbs, axis=1, keepdims=True)
    w1 = w1r[...]
    pe = jnp.tanh(ps[:, 0:1] * w1 + b1r)
    for i in range(1, 3):
        pe = pe + jnp.tanh(ps[:, i:i + 1] * w1 + b1r)
    pe = pe + 3.0 * jnp.tanh(tmp * w2r[...] + b2r)
    protector = jnp.tanh(
        jnp.dot(pe, w0t[...], preferred_element_type=jnp.float32) + b0r)

    neighbor = jnp.tanh(
        jnp.dot(ne, w5t[...], preferred_element_type=jnp.float32) + b5r)

    wc = wcb[...]
    out_ref[...] = jnp.tanh(
        protector * wc[0:1, :] + breaker * wc[1:2, :]
        + neighbor * wc[2:3, :] + wc[3:4, :])


@jax.jit
def _tc_dense(ne, cbs, ps, w0t, w1r, w2r, w3t, w4r, w5t, bias, wcb):
    grid = (N_PAD // TC_BLK,)
    full = lambda shape: pl.BlockSpec(shape, lambda i: (0, 0))
    return pl.pallas_call(
        _tc_body,
        grid=grid,
        in_specs=[
            pl.BlockSpec((TC_BLK, EMB), lambda i: (i, 0)),
            pl.BlockSpec((TC_BLK, DEG), lambda i: (i, 0)),
            pl.BlockSpec((TC_BLK, 4), lambda i: (i, 0)),
            full((EMB, EMB)), full((1, EMB)), full((1, EMB)),
            full((EMB, EMB)), full((1, EMB)), full((EMB, EMB)),
            full((8, EMB)), full((8, EMB)),
        ],
        out_specs=pl.BlockSpec((TC_BLK, EMB), lambda i: (i, 0)),
        out_shape=jax.ShapeDtypeStruct((N_PAD, EMB), jnp.float32),
    )(ne, cbs, ps, w0t, w1r, w2r, w3t, w4r, w5t, bias, wcb)


def kernel(V_pre, devices, breakers, protector_sate, breaker_state,
           W0, b0, W1, b1, W2, b2, W3, b3, W4, b4, W5, b5, Wc, bc):
    dev = devices.astype(jnp.int32)
    br = breakers.astype(jnp.int32)
    pkt = br[:, 0] | (br[:, 1] << 14)

    dev_p = jnp.pad(dev, ((0, N_PAD - N_DEV), (0, 0)))
    devt = (dev_p.T.reshape(DEG, NW, N_CH, CH)
            .transpose(1, 0, 2, 3).reshape(NW, NJ, CH))
    ps_p = jnp.pad(protector_sate, ((0, N_PAD - N_DEV), (0, 1)))

    ne, cbs3 = _sc_gather(pkt, breaker_state, devt, V_pre)
    cbs = (cbs3.reshape(NW, DEG, N_CH, CH)
           .transpose(0, 2, 3, 1).reshape(N_PAD, DEG))

    row = lambda v: v.reshape(1, EMB)
    bias = jnp.concatenate(
        [row(b0), row(b1), row(b2), row(b3), row(b4), row(b5),
         jnp.zeros((2, EMB), jnp.float32)], axis=0)
    wcb = jnp.concatenate(
        [jnp.broadcast_to(Wc[0], (1, EMB)), jnp.broadcast_to(Wc[1], (1, EMB)),
         jnp.broadcast_to(Wc[2], (1, EMB)), jnp.broadcast_to(bc[0], (1, EMB)),
         jnp.zeros((4, EMB), jnp.float32)], axis=0)

    out = _tc_dense(ne, cbs, ps_p, W0.T, W1.T, W2.T, W3.T, W4.T, W5.T,
                    bias, wcb)
    return out[:N_DEV]


# speedup vs baseline: 1.5745x; 1.4540x over previous
import functools

import jax
import jax.numpy as jnp
from jax import lax
from jax.experimental import pallas as pl
from jax.experimental.pallas import tpu as pltpu
from jax.experimental.pallas import tpu_sc as plsc

N_DEV = 10000
DEG = 16
N_BRE = 80000
EMB = 128

NW = 32
PER_W = 320
N_PAD = NW * PER_W
CH = 80
N_CH = PER_W // CH
NJ = DEG * N_CH

TC_BLK = 256


def _sc_body(pkt_hbm, bs_hbm, devt_hbm, vpre_hbm, ne_hbm, cbs_hbm,
             dev_t, tpk, tbs, nb_v, stage, acc,
             sem_pk, sem_b, sem_v):
    wid = lax.axis_index("s") * 2 + lax.axis_index("c")
    base = wid * PER_W

    pltpu.sync_copy(devt_hbm.at[wid], dev_t)

    iota16 = lax.iota(jnp.int32, 16)

    def fire_pkt(d):
        for ci in range(N_CH):
            j = d * N_CH + ci
            pltpu.async_copy(pkt_hbm.at[dev_t.at[j]], tpk.at[j], sem_pk)

    def drain_pkt():
        for _ in range(N_CH):
            pltpu.make_async_copy(
                pkt_hbm.at[pl.ds(0, CH)], tpk.at[0], sem_pk).wait()

    def fire_bs(d):
        for ci in range(N_CH):
            j = d * N_CH + ci
            pltpu.async_copy(bs_hbm.at[dev_t.at[j]], tbs.at[j], sem_b)

    def compute_nb(d):
        for ci in range(N_CH):
            j = d * N_CH + ci
            for g in range(CH // 16):
                sl = pl.ds(g * 16, 16)
                pk = tpk[j, sl]
                e0 = pk & 16383
                e1 = pk >> 14
                nid = base + ci * CH + g * 16 + iota16
                nb_v[j, sl] = jnp.where(e0 != nid, e0, e1)

    def fire_stage(d, ci):
        pltpu.async_copy(
            vpre_hbm.at[nb_v.at[d * N_CH + ci]],
            stage.at[pl.ds(ci * CH, CH)], sem_v)

    def drain_stage():
        pltpu.make_async_copy(
            vpre_hbm.at[pl.ds(0, CH)],
            stage.at[pl.ds(0, CH)], sem_v).wait()

    def acc_update(ci, init):
        def row_body(r, carry):
            for k in range(EMB // 32):
                x32 = stage[r, pl.ds(k * 32, 32)]
                lo, hi = plsc.unpack(x32, format=plsc.PackFormat.INTERLEAVED)
                sl0 = pl.ds(k * 32, 16)
                sl1 = pl.ds(k * 32 + 16, 16)
                if init:
                    acc[r, sl0] = lo
                    acc[r, sl1] = hi
                else:
                    acc[r, sl0] = acc[r, sl0] + lo
                    acc[r, sl1] = acc[r, sl1] + hi
            return carry
        lax.fori_loop(ci * CH, ci * CH + CH, row_body, 0, unroll=4)

    def acc_add(ci):
        acc_update(ci, init=False)

    fire_pkt(0)
    fire_pkt(1)
    drain_pkt()
    compute_nb(0)
    for ci in range(N_CH):
        fire_stage(0, ci)
    fire_bs(0)
    fire_pkt(2)
    drain_pkt()
    compute_nb(1)
    for ci in range(N_CH):
        drain_stage()
        acc_update(ci, init=True)
        fire_stage(1, ci)
    fire_bs(1)
    fire_pkt(3)

    def d_body(d, carry):
        drain_pkt()
        compute_nb(d + 1)
        fire_pkt(jnp.minimum(d + 3, DEG - 1))
        fire_bs(d + 1)
        for ci in range(N_CH):
            drain_stage()
            acc_add(ci)
            fire_stage(d + 1, ci)
        return carry

    lax.fori_loop(1, DEG - 1, d_body, 0, unroll=False)

    for ci in range(N_CH):
        drain_stage()
        acc_add(ci)
    drain_pkt()
    drain_pkt()

    def bs_drain_body(i, carry):
        for _ in range(N_CH):
            pltpu.make_async_copy(
                bs_hbm.at[pl.ds(0, CH)], tbs.at[0], sem_b).wait()
        return carry
    lax.fori_loop(0, DEG, bs_drain_body, 0, unroll=False)

    pltpu.sync_copy(acc, ne_hbm.at[pl.ds(base, PER_W)])
    pltpu.sync_copy(tbs, cbs_hbm.at[wid])


@jax.jit
def _sc_gather(pkt, bst, devt, vpre):
    mesh = plsc.VectorSubcoreMesh(core_axis_name="c", subcore_axis_name="s")
    fn = functools.partial(
        pl.kernel,
        out_type=(
            jax.ShapeDtypeStruct((N_PAD, EMB), jnp.float32),
            jax.ShapeDtypeStruct((NW, NJ, CH), jnp.float32),
        ),
        mesh=mesh,
        compiler_params=pltpu.CompilerParams(
            needs_layout_passes=False, use_tc_tiling_on_sc=False),
        scratch_types=[
            pltpu.VMEM((NJ, CH), jnp.int32),
            pltpu.VMEM((NJ, CH), jnp.int32),
            pltpu.VMEM((NJ, CH), jnp.float32),
            pltpu.VMEM((NJ, CH), jnp.int32),
            pltpu.VMEM((PER_W, EMB), jnp.bfloat16),
            pltpu.VMEM((PER_W, EMB), jnp.float32),
            pltpu.SemaphoreType.DMA,
            pltpu.SemaphoreType.DMA,
            pltpu.SemaphoreType.DMA,
        ],
    )(_sc_body)
    return fn(pkt, bst, devt, vpre)


def _tc_body(ne_ref, cbs_ref, ps_ref, w0t, w1r, w2r, w3t, w4r, w5t,
             bias, wcb, out_ref):
    cbs = cbs_ref[...]
    ps = ps_ref[...]
    ne = ne_ref[...]

    b0r = bias[0:1, :]
    b1r = bias[1:2, :]
    b2r = bias[2:3, :]
    b3r = bias[3:4, :]
    b4r = bias[4:5, :]
    b5r = bias[5:6, :]

    w4 = w4r[...]
    be = jnp.tanh(cbs[:, 0:1] * w4 + b4r)
    for d in range(1, DEG):
        be = be + jnp.tanh(cbs[:, d:d + 1] * w4 + b4r)
    breaker = jnp.tanh(
        jnp.dot(be, w3t[...], preferred_element_type=jnp.float32) + b3r)

    tmp = jnp.sum(cbs, axis=1, keepdims=True)
    w1 = w1r[...]
    pe = jnp.tanh(ps[:, 0:1] * w1 + b1r)
    for i in range(1, 3):
        pe = pe + jnp.tanh(ps[:, i:i + 1] * w1 + b1r)
    pe = pe + 3.0 * jnp.tanh(tmp * w2r[...] + b2r)
    protector = jnp.tanh(
        jnp.dot(pe, w0t[...], preferred_element_type=jnp.float32) + b0r)

    neighbor = jnp.tanh(
        jnp.dot(ne, w5t[...], preferred_element_type=jnp.float32) + b5r)

    wc = wcb[...]
    out_ref[...] = jnp.tanh(
        protector * wc[0:1, :] + breaker * wc[1:2, :]
        + neighbor * wc[2:3, :] + wc[3:4, :])


@jax.jit
def _tc_dense(ne, cbs, ps, w0t, w1r, w2r, w3t, w4r, w5t, bias, wcb):
    grid = (N_PAD // TC_BLK,)
    full = lambda shape: pl.BlockSpec(shape, lambda i: (0, 0))
    return pl.pallas_call(
        _tc_body,
        grid=grid,
        in_specs=[
            pl.BlockSpec((TC_BLK, EMB), lambda i: (i, 0)),
            pl.BlockSpec((TC_BLK, DEG), lambda i: (i, 0)),
            pl.BlockSpec((TC_BLK, 4), lambda i: (i, 0)),
            full((EMB, EMB)), full((1, EMB)), full((1, EMB)),
            full((EMB, EMB)), full((1, EMB)), full((EMB, EMB)),
            full((8, EMB)), full((8, EMB)),
        ],
        out_specs=pl.BlockSpec((TC_BLK, EMB), lambda i: (i, 0)),
        out_shape=jax.ShapeDtypeStruct((N_DEV, EMB), jnp.float32),
    )(ne, cbs, ps, w0t, w1r, w2r, w3t, w4r, w5t, bias, wcb)


def kernel(V_pre, devices, breakers, protector_sate, breaker_state,
           W0, b0, W1, b1, W2, b2, W3, b3, W4, b4, W5, b5, Wc, bc):
    dev = devices.astype(jnp.int32)
    br = breakers.astype(jnp.int32)
    pkt = br[:, 0] | (br[:, 1] << 14)

    dev_p = jnp.pad(dev, ((0, N_PAD - N_DEV), (0, 0)))
    devt = (dev_p.T.reshape(DEG, NW, N_CH, CH)
            .transpose(1, 0, 2, 3).reshape(NW, NJ, CH))
    ps_p = jnp.pad(protector_sate, ((0, N_PAD - N_DEV), (0, 1)))

    colperm = jnp.asarray(
        [32 * k + (c // 2 + 16 * (c & 1))
         for k in range(EMB // 32) for c in range(32)], dtype=jnp.int32)
    v16 = V_pre.astype(jnp.bfloat16)[:, colperm]

    ne, cbs3 = _sc_gather(pkt, breaker_state, devt, v16)
    cbs = (cbs3.reshape(NW, DEG, N_CH, CH)
           .transpose(0, 2, 3, 1).reshape(N_PAD, DEG))

    row = lambda v: v.reshape(1, EMB)
    bias = jnp.concatenate(
        [row(b0), row(b1), row(b2), row(b3), row(b4), row(b5),
         jnp.zeros((2, EMB), jnp.float32)], axis=0)
    wcb = jnp.concatenate(
        [jnp.broadcast_to(Wc[0], (1, EMB)), jnp.broadcast_to(Wc[1], (1, EMB)),
         jnp.broadcast_to(Wc[2], (1, EMB)), jnp.broadcast_to(bc[0], (1, EMB)),
         jnp.zeros((4, EMB), jnp.float32)], axis=0)

    return _tc_dense(ne, cbs, ps_p, W0.T, W1.T, W2.T, W3.T, W4.T, W5.T,
                     bias, wcb)


# speedup vs baseline: 1.6196x; 1.0286x over previous
import functools

import jax
import jax.numpy as jnp
from jax import lax
from jax.experimental import pallas as pl
from jax.experimental.pallas import tpu as pltpu
from jax.experimental.pallas import tpu_sc as plsc

N_DEV = 10000
DEG = 16
N_BRE = 80000
EMB = 128

NW = 32
PER_W = 320
N_PAD = NW * PER_W
CH = 80
N_CH = PER_W // CH
NJ = DEG * N_CH

TC_BLK = 256


def _sc_body(pkt_hbm, bs_hbm, devt_hbm, vpre_hbm, ne_hbm, cbs_hbm,
             dev_t, tpk, tbs, nb_v, stage, acc,
             sem_pk, sem_b, sem_v):
    wid = lax.axis_index("s") * 2 + lax.axis_index("c")
    base = wid * PER_W

    pltpu.sync_copy(devt_hbm.at[wid], dev_t)

    iota16 = lax.iota(jnp.int32, 16)

    def fire_pkt(d):
        for ci in range(N_CH):
            j = d * N_CH + ci
            pltpu.async_copy(pkt_hbm.at[dev_t.at[j]], tpk.at[j], sem_pk)

    def drain_pkt():
        for _ in range(N_CH):
            pltpu.make_async_copy(
                pkt_hbm.at[pl.ds(0, CH)], tpk.at[0], sem_pk).wait()

    def fire_bs(d):
        for ci in range(N_CH):
            j = d * N_CH + ci
            pltpu.async_copy(bs_hbm.at[dev_t.at[j]], tbs.at[j], sem_b)

    def compute_nb(d):
        for ci in range(N_CH):
            j = d * N_CH + ci
            for g in range(CH // 16):
                sl = pl.ds(g * 16, 16)
                pk = tpk[j, sl]
                e0 = pk & 16383
                e1 = pk >> 14
                nid = base + ci * CH + g * 16 + iota16
                nb_v[j, sl] = jnp.where(e0 != nid, e0, e1)

    def fire_stage(d, ci):
        pltpu.async_copy(
            vpre_hbm.at[nb_v.at[d * N_CH + ci]],
            stage.at[pl.ds(ci * CH, CH)], sem_v)

    def drain_stage():
        pltpu.make_async_copy(
            vpre_hbm.at[pl.ds(0, CH)],
            stage.at[pl.ds(0, CH)], sem_v).wait()

    def acc_update(ci, init):
        def row_body(r, carry):
            for k in range(EMB // 32):
                x32 = stage[r, pl.ds(k * 32, 32)]
                lo, hi = plsc.unpack(x32, format=plsc.PackFormat.INTERLEAVED)
                sl0 = pl.ds(k * 32, 16)
                sl1 = pl.ds(k * 32 + 16, 16)
                if init:
                    acc[r, sl0] = lo
                    acc[r, sl1] = hi
                else:
                    acc[r, sl0] = acc[r, sl0] + lo
                    acc[r, sl1] = acc[r, sl1] + hi
            return carry
        lax.fori_loop(ci * CH, ci * CH + CH, row_body, 0, unroll=4)

    def acc_add(ci):
        acc_update(ci, init=False)

    fire_pkt(0)
    fire_pkt(1)
    drain_pkt()
    compute_nb(0)
    for ci in range(N_CH):
        fire_stage(0, ci)
    fire_bs(0)
    fire_pkt(2)
    drain_pkt()
    compute_nb(1)
    for ci in range(N_CH):
        drain_stage()
        acc_update(ci, init=True)
        fire_stage(1, ci)
    fire_bs(1)
    fire_pkt(3)

    def d_body(d, carry):
        drain_pkt()
        compute_nb(d + 1)
        fire_pkt(jnp.minimum(d + 3, DEG - 1))
        fire_bs(d + 1)
        for ci in range(N_CH):
            drain_stage()
            acc_add(ci)
            fire_stage(d + 1, ci)
        return carry

    lax.fori_loop(1, DEG - 1, d_body, 0, unroll=False)

    for ci in range(N_CH):
        drain_stage()
        acc_add(ci)
    drain_pkt()
    drain_pkt()

    def bs_drain_body(i, carry):
        for _ in range(N_CH):
            pltpu.make_async_copy(
                bs_hbm.at[pl.ds(0, CH)], tbs.at[0], sem_b).wait()
        return carry
    lax.fori_loop(0, DEG, bs_drain_body, 0, unroll=False)

    pltpu.sync_copy(acc, ne_hbm.at[pl.ds(base, PER_W)])
    pltpu.sync_copy(tbs, cbs_hbm.at[wid])


@jax.jit
def _sc_gather(pkt, bst, devt, vpre):
    mesh = plsc.VectorSubcoreMesh(core_axis_name="c", subcore_axis_name="s")
    fn = functools.partial(
        pl.kernel,
        out_type=(
            jax.ShapeDtypeStruct((N_PAD, EMB), jnp.float32),
            jax.ShapeDtypeStruct((NW, NJ, CH), jnp.float32),
        ),
        mesh=mesh,
        compiler_params=pltpu.CompilerParams(
            needs_layout_passes=False, use_tc_tiling_on_sc=False),
        scratch_types=[
            pltpu.VMEM((NJ, CH), jnp.int32),
            pltpu.VMEM((NJ, CH), jnp.int32),
            pltpu.VMEM((NJ, CH), jnp.float32),
            pltpu.VMEM((NJ, CH), jnp.int32),
            pltpu.VMEM((PER_W, EMB), jnp.bfloat16),
            pltpu.VMEM((PER_W, EMB), jnp.float32),
            pltpu.SemaphoreType.DMA,
            pltpu.SemaphoreType.DMA,
            pltpu.SemaphoreType.DMA,
        ],
    )(_sc_body)
    return fn(pkt, bst, devt, vpre)


def _tc_body(ne_ref, cbs_ref, ps_ref, w0t, w1r, w2r, w3t, w4r, w5t,
             bias, wcb, out_ref):
    cbs = cbs_ref[...]
    ps = ps_ref[...]
    ne = ne_ref[...]

    b0r = bias[0:1, :]
    b1r = bias[1:2, :]
    b2r = bias[2:3, :]
    b3r = bias[3:4, :]
    b4r = bias[4:5, :]
    b5r = bias[5:6, :]

    w4 = w4r[...]
    be = jnp.tanh(cbs[:, 0:1] * w4 + b4r)
    for d in range(1, DEG):
        be = be + jnp.tanh(cbs[:, d:d + 1] * w4 + b4r)
    breaker = jnp.tanh(
        jnp.dot(be, w3t[...], preferred_element_type=jnp.float32) + b3r)

    tmp = jnp.sum(cbs, axis=1, keepdims=True)
    w1 = w1r[...]
    pe = jnp.tanh(ps[:, 0:1] * w1 + b1r)
    for i in range(1, 3):
        pe = pe + jnp.tanh(ps[:, i:i + 1] * w1 + b1r)
    pe = pe + 3.0 * jnp.tanh(tmp * w2r[...] + b2r)
    protector = jnp.tanh(
        jnp.dot(pe, w0t[...], preferred_element_type=jnp.float32) + b0r)

    neighbor = jnp.tanh(
        jnp.dot(ne, w5t[...], preferred_element_type=jnp.float32) + b5r)

    wc = wcb[...]
    out_ref[...] = jnp.tanh(
        protector * wc[0:1, :] + breaker * wc[1:2, :]
        + neighbor * wc[2:3, :] + wc[3:4, :])


@jax.jit
def _tc_dense(ne, cbs, ps, w0t, w1r, w2r, w3t, w4r, w5t, bias, wcb):
    grid = (N_PAD // TC_BLK,)
    full = lambda shape: pl.BlockSpec(shape, lambda i: (0, 0))
    return pl.pallas_call(
        _tc_body,
        grid=grid,
        in_specs=[
            pl.BlockSpec((TC_BLK, EMB), lambda i: (i, 0)),
            pl.BlockSpec((TC_BLK, DEG), lambda i: (i, 0)),
            pl.BlockSpec((TC_BLK, 4), lambda i: (i, 0)),
            full((EMB, EMB)), full((1, EMB)), full((1, EMB)),
            full((EMB, EMB)), full((1, EMB)), full((EMB, EMB)),
            full((8, EMB)), full((8, EMB)),
        ],
        out_specs=pl.BlockSpec((TC_BLK, EMB), lambda i: (i, 0)),
        out_shape=jax.ShapeDtypeStruct((N_DEV, EMB), jnp.float32),
    )(ne, cbs, ps, w0t, w1r, w2r, w3t, w4r, w5t, bias, wcb)


def kernel(V_pre, devices, breakers, protector_sate, breaker_state,
           W0, b0, W1, b1, W2, b2, W3, b3, W4, b4, W5, b5, Wc, bc):
    dev = devices.astype(jnp.int32)
    br = breakers.astype(jnp.int32)
    pkt = br[:, 0] | (br[:, 1] << 14)

    dev_p = jnp.pad(dev, ((0, N_PAD - N_DEV), (0, 0)))
    devt = (dev_p.T.reshape(DEG, NW, N_CH, CH)
            .transpose(1, 0, 2, 3).reshape(NW, NJ, CH))
    ps_p = jnp.pad(protector_sate, ((0, N_PAD - N_DEV), (0, 1)))

    q = [32 * (c // 32) + 2 * (c % 32) if (c % 32) < 16
         else 32 * (c // 32) + 2 * ((c % 32) - 16) + 1 for c in range(EMB)]
    w5tp = W5.T[jnp.asarray(q, dtype=jnp.int32), :]
    v16 = V_pre.astype(jnp.bfloat16)

    ne, cbs3 = _sc_gather(pkt, breaker_state, devt, v16)
    cbs = (cbs3.reshape(NW, DEG, N_CH, CH)
           .transpose(0, 2, 3, 1).reshape(N_PAD, DEG))

    row = lambda v: v.reshape(1, EMB)
    bias = jnp.concatenate(
        [row(b0), row(b1), row(b2), row(b3), row(b4), row(b5),
         jnp.zeros((2, EMB), jnp.float32)], axis=0)
    wcb = jnp.concatenate(
        [jnp.broadcast_to(Wc[0], (1, EMB)), jnp.broadcast_to(Wc[1], (1, EMB)),
         jnp.broadcast_to(Wc[2], (1, EMB)), jnp.broadcast_to(bc[0], (1, EMB)),
         jnp.zeros((4, EMB), jnp.float32)], axis=0)

    return _tc_dense(ne, cbs, ps_p, W0.T, W1.T, W2.T, W3.T, W4.T, w5tp,
                     bias, wcb)


# speedup vs baseline: 1.6284x; 1.0054x over previous
import functools

import jax
import jax.numpy as jnp
from jax import lax
from jax.experimental import pallas as pl
from jax.experimental.pallas import tpu as pltpu
from jax.experimental.pallas import tpu_sc as plsc

N_DEV = 10000
DEG = 16
N_BRE = 80000
EMB = 128

NW = 32
PER_W = 320
N_PAD = NW * PER_W
CH = 80
N_CH = PER_W // CH
NJ = DEG * N_CH

TC_BLK = 256


def _sc_body(pkt_hbm, bs_hbm, devt_hbm, vpre_hbm, ne_hbm, cbs_hbm,
             dev_t, tpk, tbs, nb_v, stage, acc,
             sem_pk, sem_b, sem_v, sem_o):
    wid = lax.axis_index("s") * 2 + lax.axis_index("c")
    base = wid * PER_W

    pltpu.sync_copy(devt_hbm.at[wid], dev_t)

    iota16 = lax.iota(jnp.int32, 16)

    def fire_pkt(d):
        for ci in range(N_CH):
            j = d * N_CH + ci
            pltpu.async_copy(pkt_hbm.at[dev_t.at[j]], tpk.at[j], sem_pk)

    def drain_pkt():
        for _ in range(N_CH):
            pltpu.make_async_copy(
                pkt_hbm.at[pl.ds(0, CH)], tpk.at[0], sem_pk).wait()

    def fire_bs(d):
        for ci in range(N_CH):
            j = d * N_CH + ci
            pltpu.async_copy(bs_hbm.at[dev_t.at[j]], tbs.at[j], sem_b)

    def compute_nb(d):
        for ci in range(N_CH):
            j = d * N_CH + ci
            for g in range(CH // 16):
                sl = pl.ds(g * 16, 16)
                pk = tpk[j, sl]
                e0 = pk & 16383
                e1 = pk >> 14
                nid = base + ci * CH + g * 16 + iota16
                nb_v[j, sl] = jnp.where(e0 != nid, e0, e1)

    def fire_stage(d, ci):
        pltpu.async_copy(
            vpre_hbm.at[nb_v.at[d * N_CH + ci]],
            stage.at[pl.ds(ci * CH, CH)], sem_v)

    def drain_stage():
        pltpu.make_async_copy(
            vpre_hbm.at[pl.ds(0, CH)],
            stage.at[pl.ds(0, CH)], sem_v).wait()

    def acc_update(ci, init):
        def row_body(r, carry):
            for k in range(EMB // 32):
                x32 = stage[r, pl.ds(k * 32, 32)]
                lo, hi = plsc.unpack(x32, format=plsc.PackFormat.INTERLEAVED)
                sl0 = pl.ds(k * 32, 16)
                sl1 = pl.ds(k * 32 + 16, 16)
                if init:
                    acc[r, sl0] = lo
                    acc[r, sl1] = hi
                else:
                    acc[r, sl0] = acc[r, sl0] + lo
                    acc[r, sl1] = acc[r, sl1] + hi
            return carry
        lax.fori_loop(ci * CH, ci * CH + CH, row_body, 0, unroll=4)

    def acc_add(ci):
        acc_update(ci, init=False)

    fire_pkt(0)
    fire_pkt(1)
    drain_pkt()
    compute_nb(0)
    for ci in range(N_CH):
        fire_stage(0, ci)
    fire_bs(0)
    fire_pkt(2)
    drain_pkt()
    compute_nb(1)
    for ci in range(N_CH):
        drain_stage()
        acc_update(ci, init=True)
        fire_stage(1, ci)
    fire_bs(1)
    fire_pkt(3)

    def drain_bs():
        for _ in range(N_CH):
            pltpu.make_async_copy(
                bs_hbm.at[pl.ds(0, CH)], tbs.at[0], sem_b).wait()

    def d_body(d, carry):
        drain_pkt()
        compute_nb(d + 1)
        fire_pkt(jnp.minimum(d + 3, DEG - 1))
        drain_bs()
        fire_bs(d + 1)
        for ci in range(N_CH):
            drain_stage()
            acc_add(ci)
            fire_stage(d + 1, ci)
        return carry

    lax.fori_loop(1, DEG - 1, d_body, 0, unroll=False)

    for ci in range(N_CH):
        drain_stage()
        acc_add(ci)
        pltpu.async_copy(acc.at[pl.ds(ci * CH, CH)],
                         ne_hbm.at[pl.ds(base + ci * CH, CH)], sem_o)
    drain_pkt()
    drain_pkt()
    drain_bs()
    drain_bs()
    pltpu.async_copy(tbs, cbs_hbm.at[wid], sem_o)
    for ci in range(N_CH):
        pltpu.make_async_copy(acc.at[pl.ds(0, CH)],
                              ne_hbm.at[pl.ds(0, CH)], sem_o).wait()
    pltpu.make_async_copy(tbs, cbs_hbm.at[0], sem_o).wait()


@jax.jit
def _sc_gather(pkt, bst, devt, vpre):
    mesh = plsc.VectorSubcoreMesh(core_axis_name="c", subcore_axis_name="s")
    fn = functools.partial(
        pl.kernel,
        out_type=(
            jax.ShapeDtypeStruct((N_PAD, EMB), jnp.float32),
            jax.ShapeDtypeStruct((NW, NJ, CH), jnp.float32),
        ),
        mesh=mesh,
        compiler_params=pltpu.CompilerParams(
            needs_layout_passes=False, use_tc_tiling_on_sc=False),
        scratch_types=[
            pltpu.VMEM((NJ, CH), jnp.int32),
            pltpu.VMEM((NJ, CH), jnp.int32),
            pltpu.VMEM((NJ, CH), jnp.float32),
            pltpu.VMEM((NJ, CH), jnp.int32),
            pltpu.VMEM((PER_W, EMB), jnp.bfloat16),
            pltpu.VMEM((PER_W, EMB), jnp.float32),
            pltpu.SemaphoreType.DMA,
            pltpu.SemaphoreType.DMA,
            pltpu.SemaphoreType.DMA,
            pltpu.SemaphoreType.DMA,
        ],
    )(_sc_body)
    return fn(pkt, bst, devt, vpre)


def _tc_body(ne_ref, cbs_ref, ps_ref, w0t, w1r, w2r, w3t, w4r, w5t,
             bias, wcb, out_ref):
    cbs = cbs_ref[...]
    ps = ps_ref[...]
    ne = ne_ref[...]

    b0r = bias[0:1, :]
    b1r = bias[1:2, :]
    b2r = bias[2:3, :]
    b3r = bias[3:4, :]
    b4r = bias[4:5, :]
    b5r = bias[5:6, :]

    w4 = w4r[...]
    be = jnp.tanh(cbs[:, 0:1] * w4 + b4r)
    for d in range(1, DEG):
        be = be + jnp.tanh(cbs[:, d:d + 1] * w4 + b4r)
    breaker = jnp.tanh(
        jnp.dot(be, w3t[...], preferred_element_type=jnp.float32) + b3r)

    tmp = jnp.sum(cbs, axis=1, keepdims=True)
    w1 = w1r[...]
    pe = jnp.tanh(ps[:, 0:1] * w1 + b1r)
    for i in range(1, 3):
        pe = pe + jnp.tanh(ps[:, i:i + 1] * w1 + b1r)
    pe = pe + 3.0 * jnp.tanh(tmp * w2r[...] + b2r)
    protector = jnp.tanh(
        jnp.dot(pe, w0t[...], preferred_element_type=jnp.float32) + b0r)

    neighbor = jnp.tanh(
        jnp.dot(ne, w5t[...], preferred_element_type=jnp.float32) + b5r)

    wc = wcb[...]
    out_ref[...] = jnp.tanh(
        protector * wc[0:1, :] + breaker * wc[1:2, :]
        + neighbor * wc[2:3, :] + wc[3:4, :])


@jax.jit
def _tc_dense(ne, cbs, ps, w0t, w1r, w2r, w3t, w4r, w5t, bias, wcb):
    grid = (N_PAD // TC_BLK,)
    full = lambda shape: pl.BlockSpec(shape, lambda i: (0, 0))
    return pl.pallas_call(
        _tc_body,
        grid=grid,
        in_specs=[
            pl.BlockSpec((TC_BLK, EMB), lambda i: (i, 0)),
            pl.BlockSpec((TC_BLK, DEG), lambda i: (i, 0)),
            pl.BlockSpec((TC_BLK, 4), lambda i: (i, 0)),
            full((EMB, EMB)), full((1, EMB)), full((1, EMB)),
            full((EMB, EMB)), full((1, EMB)), full((EMB, EMB)),
            full((8, EMB)), full((8, EMB)),
        ],
        out_specs=pl.BlockSpec((TC_BLK, EMB), lambda i: (i, 0)),
        out_shape=jax.ShapeDtypeStruct((N_DEV, EMB), jnp.float32),
    )(ne, cbs, ps, w0t, w1r, w2r, w3t, w4r, w5t, bias, wcb)


def kernel(V_pre, devices, breakers, protector_sate, breaker_state,
           W0, b0, W1, b1, W2, b2, W3, b3, W4, b4, W5, b5, Wc, bc):
    dev = devices.astype(jnp.int32)
    br = breakers.astype(jnp.int32)
    pkt = br[:, 0] | (br[:, 1] << 14)

    dev_p = jnp.pad(dev, ((0, N_PAD - N_DEV), (0, 0)))
    devt = (dev_p.T.reshape(DEG, NW, N_CH, CH)
            .transpose(1, 0, 2, 3).reshape(NW, NJ, CH))
    ps_p = jnp.pad(protector_sate, ((0, N_PAD - N_DEV), (0, 1)))

    q = [32 * (c // 32) + 2 * (c % 32) if (c % 32) < 16
         else 32 * (c // 32) + 2 * ((c % 32) - 16) + 1 for c in range(EMB)]
    w5tp = W5.T[jnp.asarray(q, dtype=jnp.int32), :]
    v16 = V_pre.astype(jnp.bfloat16)

    ne, cbs3 = _sc_gather(pkt, breaker_state, devt, v16)
    cbs = (cbs3.reshape(NW, DEG, N_CH, CH)
           .transpose(0, 2, 3, 1).reshape(N_PAD, DEG))

    row = lambda v: v.reshape(1, EMB)
    bias = jnp.concatenate(
        [row(b0), row(b1), row(b2), row(b3), row(b4), row(b5),
         jnp.zeros((2, EMB), jnp.float32)], axis=0)
    wcb = jnp.concatenate(
        [jnp.broadcast_to(Wc[0], (1, EMB)), jnp.broadcast_to(Wc[1], (1, EMB)),
         jnp.broadcast_to(Wc[2], (1, EMB)), jnp.broadcast_to(bc[0], (1, EMB)),
         jnp.zeros((4, EMB), jnp.float32)], axis=0)

    return _tc_dense(ne, cbs, ps_p, W0.T, W1.T, W2.T, W3.T, W4.T, w5tp,
                     bias, wcb)
